# speedup vs baseline: 1.5190x; 1.5190x over previous
import functools

import jax
import jax.numpy as jnp
from jax.experimental import pallas as pl
from jax.experimental.pallas import tpu as pltpu

T, D_MODEL, D_FF, E, TOP_K = 2048, 768, 2048, 8, 2
EPAD = 128
NEG = -1e30


def _routing_body(x_ref, wg_ref, steer_ref, c_ref):
    logits = jnp.dot(x_ref[...], wg_ref[...], preferred_element_type=jnp.float32)
    logits = logits + steer_ref[...]
    lane = jax.lax.broadcasted_iota(jnp.int32, (T, EPAD), 1)
    m1 = jnp.max(logits, axis=1, keepdims=True)
    i1 = jnp.min(jnp.where(logits == m1, lane, EPAD), axis=1, keepdims=True)
    masked = jnp.where(lane == i1, NEG, logits)
    m2 = jnp.max(masked, axis=1, keepdims=True)
    i2 = jnp.min(jnp.where(masked == m2, lane, EPAD), axis=1, keepdims=True)
    w0 = 1.0 / (1.0 + jnp.exp(m2 - m1))
    w1 = 1.0 - w0
    c_ref[...] = jnp.where(lane == i1, w0, 0.0) + jnp.where(lane == i2, w1, 0.0)


def _ffn_body(c_ref, x_ref, w1_ref, w3_ref, w2_ref, out_ref):
    e = pl.program_id(0)
    f = pl.program_id(1)
    xb = x_ref[...]
    a = jnp.dot(xb, w1_ref[0], preferred_element_type=jnp.float32)
    b = jnp.dot(xb, w3_ref[0], preferred_element_type=jnp.float32)
    h = (a * jax.nn.sigmoid(a)) * b
    p = jnp.dot(h, w2_ref[0], preferred_element_type=jnp.float32)
    lane = jax.lax.broadcasted_iota(jnp.int32, (T, EPAD), 1)
    c_col = jnp.sum(jnp.where(lane == e, c_ref[...], 0.0), axis=1, keepdims=True)
    contrib = c_col * p

    @pl.when((e == 0) & (f == 0))
    def _init():
        out_ref[...] = contrib

    @pl.when((e > 0) | (f > 0))
    def _acc():
        out_ref[...] += contrib


def kernel(hidden_states, Wg, steer_vector, W1, W3, W2):
    x = hidden_states
    wg_pad = jnp.zeros((D_MODEL, EPAD), jnp.float32).at[:, :E].set(Wg)
    steer_pad = jnp.full((1, EPAD), NEG, jnp.float32).at[0, :E].set(steer_vector)

    c = pl.pallas_call(
        _routing_body,
        out_shape=jax.ShapeDtypeStruct((T, EPAD), jnp.float32),
    )(x, wg_pad, steer_pad)

    FB = 512
    NF = D_FF // FB
    out = pl.pallas_call(
        _ffn_body,
        grid=(E, NF),
        in_specs=[
            pl.BlockSpec((T, EPAD), lambda e, f: (0, 0)),
            pl.BlockSpec((T, D_MODEL), lambda e, f: (0, 0)),
            pl.BlockSpec((1, D_MODEL, FB), lambda e, f: (e, 0, f)),
            pl.BlockSpec((1, D_MODEL, FB), lambda e, f: (e, 0, f)),
            pl.BlockSpec((1, FB, D_MODEL), lambda e, f: (e, f, 0)),
        ],
        out_specs=pl.BlockSpec((T, D_MODEL), lambda e, f: (0, 0)),
        out_shape=jax.ShapeDtypeStruct((T, D_MODEL), jnp.float32),
        compiler_params=pltpu.CompilerParams(
            dimension_semantics=("arbitrary", "arbitrary"),
        ),
    )(c, x, W1, W3, W2)
    return out
